# baseline (speedup 1.0000x reference)
import jax
import jax.numpy as jnp
from jax.experimental import pallas as pl
from jax.experimental.pallas import tpu as pltpu
from jax.experimental.pallas import tpu_sc as plsc

_NUM_CORES = 2
_NUM_SUBCORES = 16
_CHUNK_ROWS = 32


def kernel(input, pe_table):
    length = input.shape[1]
    max_len, d = pe_table.shape
    units = _NUM_CORES * _NUM_SUBCORES
    rows_per_unit = length // units
    nblk = rows_per_unit // _CHUNK_ROWS

    mesh = plsc.VectorSubcoreMesh(core_axis_name="core", subcore_axis_name="subcore")

    @pl.kernel(out_type=jax.ShapeDtypeStruct((length, d), pe_table.dtype),
               mesh=mesh,
               scratch_types=[pltpu.VMEM((2, _CHUNK_ROWS, d), pe_table.dtype),
                              pltpu.SemaphoreType.DMA((2,)),
                              pltpu.SemaphoreType.DMA((2,))])
    def sc_gather_rows(pe_hbm, o_hbm, buf, in_sem, out_sem):
        core = jax.lax.axis_index("core")
        sub = jax.lax.axis_index("subcore")
        base = (core * _NUM_SUBCORES + sub) * rows_per_unit

        def rd(i):
            s = i % 2
            return pltpu.make_async_copy(
                pe_hbm.at[pl.ds(base + i * _CHUNK_ROWS, _CHUNK_ROWS)],
                buf.at[s], in_sem.at[s])

        def wr(i):
            s = i % 2
            return pltpu.make_async_copy(
                buf.at[s],
                o_hbm.at[pl.ds(base + i * _CHUNK_ROWS, _CHUNK_ROWS)],
                out_sem.at[s])

        rd(0).start()
        if nblk > 1:
            rd(1).start()
        for i in range(nblk):
            rd(i).wait()
            wr(i).start()
            if i + 2 < nblk:
                wr(i).wait()
                rd(i + 2).start()
        for i in range(max(0, nblk - 2), nblk):
            wr(i).wait()

    return sc_gather_rows(pe_table)[None]
